# baseline (speedup 1.0000x reference)
import jax
import jax.numpy as jnp
from jax.experimental import pallas as pl
from jax.experimental.pallas import tpu as pltpu

_DATA = 128
_HID = 6
_NM = 4
_TB = 8192


def _bulk_copy(x_hbm, xbuf, sems, step, slot):
    return pltpu.make_async_copy(
        x_hbm.at[pl.ds(step * _TB, _TB), 0:_DATA],
        xbuf.at[slot, :, 0:_DATA], sems.at[0, slot])


def _tail_copy(x_hbm, xbuf, sems, step, slot):
    return pltpu.make_async_copy(
        x_hbm.at[pl.ds(step * _TB, _TB), _DATA:_DATA + 2],
        xbuf.at[slot, :, _DATA:_DATA + 2], sems.at[1, slot])


def _start_copies(x_hbm, xbuf, sems, step, slot):
    _tail_copy(x_hbm, xbuf, sems, step, slot).start()
    _bulk_copy(x_hbm, xbuf, sems, step, slot).start()


def _wait_copies(x_hbm, xbuf, sems, step, slot):
    _bulk_copy(x_hbm, xbuf, sems, step, slot).wait()
    _tail_copy(x_hbm, xbuf, sems, step, slot).wait()


def _fused(x_hbm, w1_ref, b1_ref, w2_ref, b2_ref, o_ref,
           xbuf, w1p, b1c, m2, sems):
    i = pl.program_id(0)
    nsteps = pl.num_programs(0)
    slot = jax.lax.rem(i, 2)
    nxt = jax.lax.rem(i + 1, 2)

    @pl.when(i == 0)
    def _():
        _start_copies(x_hbm, xbuf, sems, 0, 0)
        for m in range(_NM):
            c = _HID * m
            w1p[0:_DATA, c:c + _HID] = w1_ref[m, 0:_DATA, :]
            w1p[_DATA:_DATA + 1, c:c + _HID] = jnp.zeros(
                (1, _HID), jnp.float32)
            w1p[_DATA + 1:_DATA + 2, c:c + _HID] = w1_ref[
                m, _DATA:_DATA + 1, :]
            b1c[0:1, c:c + _HID] = b1_ref[m:m + 1, :]
            m2[c:c + _HID, :] = w2_ref[m]
        m2[_NM * _HID:_NM * _HID + _NM, :] = b2_ref[...]
        m2[_NM * _HID + _NM:, :] = jnp.zeros((_NM, _DATA), jnp.float32)

    @pl.when(i + 1 < nsteps)
    def _():
        _start_copies(x_hbm, xbuf, sems, i + 1, nxt)

    _wait_copies(x_hbm, xbuf, sems, i, slot)

    x = xbuf[slot]
    cond = x[:, _DATA:_DATA + 1]
    t = x[:, _DATA + 1:_DATA + 2]
    r = jnp.where(
        cond == 0.0,
        0,
        jnp.where(cond == 1.0, jnp.where(t < 0.5, 1, 2), 3),
    )

    h = jnp.tanh(
        jax.lax.dot_general(x, w1p[...], (((1,), (0,)), ((), ())),
                            preferred_element_type=jnp.float32)
        + b1c[...]
    )

    a = jnp.concatenate([h, jnp.ones((h.shape[0], 8), jnp.float32)], axis=1)
    col = jax.lax.broadcasted_iota(jnp.int32, a.shape, 1)
    expert = jnp.where(col < _NM * _HID, col // _HID, col - _NM * _HID)
    a = a * (expert == r).astype(jnp.float32)

    o_ref[...] = jax.lax.dot_general(a, m2[...], (((1,), (0,)), ((), ())),
                                     preferred_element_type=jnp.float32)


def kernel(x, W1, b1, W2, b2):
    B = x.shape[0]
    grid = (B // _TB,)
    return pl.pallas_call(
        _fused,
        grid=grid,
        in_specs=[
            pl.BlockSpec(memory_space=pl.ANY),
            pl.BlockSpec((_NM, _DATA + 1, _HID), lambda i: (0, 0, 0)),
            pl.BlockSpec((_NM, _HID), lambda i: (0, 0)),
            pl.BlockSpec((_NM, _HID, _DATA), lambda i: (0, 0, 0)),
            pl.BlockSpec((_NM, _DATA), lambda i: (0, 0)),
        ],
        out_specs=pl.BlockSpec((_TB, _DATA), lambda i: (i, 0)),
        out_shape=jax.ShapeDtypeStruct((B, _DATA), jnp.float32),
        scratch_shapes=[
            pltpu.VMEM((2, _TB, _DATA + 2), jnp.float32),
            pltpu.VMEM((_DATA + 2, _NM * _HID), jnp.float32),
            pltpu.VMEM((1, _NM * _HID), jnp.float32),
            pltpu.VMEM((_NM * _HID + 2 * _NM, _DATA), jnp.float32),
            pltpu.SemaphoreType.DMA((2, 2)),
        ],
    )(x, W1, b1, W2, b2)


# speedup vs baseline: 1.0072x; 1.0072x over previous
import jax
import jax.numpy as jnp
from jax.experimental import pallas as pl
from jax.experimental.pallas import tpu as pltpu

_DATA = 128
_HID = 6
_NM = 4
_TB = 8192


def _tile_copy(x_hbm, xbuf, sems, step, slot):
    return pltpu.make_async_copy(
        x_hbm.at[pl.ds(step * _TB, _TB), :], xbuf.at[slot], sems.at[slot])


def _fused(x_hbm, w1_ref, b1_ref, w2_ref, b2_ref, o_ref,
           xbuf, w1p, b1c, m2, sems):
    i = pl.program_id(0)
    nsteps = pl.num_programs(0)
    slot = jax.lax.rem(i, 2)
    nxt = jax.lax.rem(i + 1, 2)

    @pl.when(i == 0)
    def _():
        _tile_copy(x_hbm, xbuf, sems, 0, 0).start()
        for m in range(_NM):
            c = _HID * m
            w1p[0:_DATA, c:c + _HID] = w1_ref[m, 0:_DATA, :]
            w1p[_DATA:_DATA + 1, c:c + _HID] = jnp.zeros(
                (1, _HID), jnp.float32)
            w1p[_DATA + 1:_DATA + 2, c:c + _HID] = w1_ref[
                m, _DATA:_DATA + 1, :]
            b1c[0:1, c:c + _HID] = b1_ref[m:m + 1, :]
            m2[c:c + _HID, :] = w2_ref[m]
        m2[_NM * _HID:_NM * _HID + _NM, :] = b2_ref[...]
        m2[_NM * _HID + _NM:, :] = jnp.zeros((_NM, _DATA), jnp.float32)

    @pl.when(i + 1 < nsteps)
    def _():
        _tile_copy(x_hbm, xbuf, sems, i + 1, nxt).start()

    _tile_copy(x_hbm, xbuf, sems, i, slot).wait()

    x = xbuf[slot]
    cond = x[:, _DATA:_DATA + 1]
    t = x[:, _DATA + 1:_DATA + 2]
    r = jnp.where(
        cond == 0.0,
        0,
        jnp.where(cond == 1.0, jnp.where(t < 0.5, 1, 2), 3),
    )

    h = jnp.tanh(
        jax.lax.dot_general(x, w1p[...], (((1,), (0,)), ((), ())),
                            preferred_element_type=jnp.float32)
        + b1c[...]
    )

    a = jnp.concatenate([h, jnp.ones((h.shape[0], 8), jnp.float32)], axis=1)
    col = jax.lax.broadcasted_iota(jnp.int32, a.shape, 1)
    expert = jnp.where(col < _NM * _HID, col // _HID, col - _NM * _HID)
    a = a * (expert == r).astype(jnp.float32)

    o_ref[...] = jax.lax.dot_general(a, m2[...], (((1,), (0,)), ((), ())),
                                     preferred_element_type=jnp.float32)


def kernel(x, W1, b1, W2, b2):
    B = x.shape[0]
    grid = (B // _TB,)
    return pl.pallas_call(
        _fused,
        grid=grid,
        in_specs=[
            pl.BlockSpec(memory_space=pl.ANY),
            pl.BlockSpec((_NM, _DATA + 1, _HID), lambda i: (0, 0, 0)),
            pl.BlockSpec((_NM, _HID), lambda i: (0, 0)),
            pl.BlockSpec((_NM, _HID, _DATA), lambda i: (0, 0, 0)),
            pl.BlockSpec((_NM, _DATA), lambda i: (0, 0)),
        ],
        out_specs=pl.BlockSpec((_TB, _DATA), lambda i: (i, 0)),
        out_shape=jax.ShapeDtypeStruct((B, _DATA), jnp.float32),
        scratch_shapes=[
            pltpu.VMEM((2, _TB, _DATA + 2), jnp.float32),
            pltpu.VMEM((_DATA + 2, _NM * _HID), jnp.float32),
            pltpu.VMEM((1, _NM * _HID), jnp.float32),
            pltpu.VMEM((_NM * _HID + 2 * _NM, _DATA), jnp.float32),
            pltpu.SemaphoreType.DMA((2,)),
        ],
    )(x, W1, b1, W2, b2)
